# baseline (speedup 1.0000x reference)
import functools

import jax
import jax.numpy as jnp
from jax import lax
from jax.experimental import pallas as pl
from jax.experimental.pallas import tpu as pltpu
from jax.experimental.pallas import tpu_sc as plsc

ENT = 100000
REL = 1000
DIM = 128
VIS = 4096
B = 4096

NC = 2
NS = 16
NW = NC * NS
CH = 8
D = 3

G = 4
CB = B // G
TB = 512


def _sc_gather(batch_h, batch_t, batch_r, ent_emb, rel_emb, vis_emb):
    nb = batch_h.shape[0]
    ROWS_W = nb // NW
    NCH = ROWS_W // CH
    mesh = plsc.VectorSubcoreMesh(core_axis_name="c", subcore_axis_name="s")

    @functools.partial(
        pl.kernel,
        out_type=(
            jax.ShapeDtypeStruct((nb, VIS), jnp.float32),
            jax.ShapeDtypeStruct((nb, VIS), jnp.float32),
            jax.ShapeDtypeStruct((nb, DIM), jnp.float32),
            jax.ShapeDtypeStruct((nb, DIM), jnp.float32),
            jax.ShapeDtypeStruct((nb, DIM), jnp.float32),
        ),
        mesh=mesh,
        scratch_types=[
            pltpu.VMEM((ROWS_W,), jnp.int32),
            pltpu.VMEM((ROWS_W,), jnp.int32),
            pltpu.VMEM((ROWS_W,), jnp.int32),
            pltpu.VMEM((D, CH, VIS), jnp.float32),
            pltpu.VMEM((ROWS_W, DIM), jnp.float32),
        ] + [pltpu.SemaphoreType.DMA] * (2 * D + 1),
    )
    def k(h_hbm, t_hbm, r_hbm, ent_hbm, rel_hbm, vis_hbm,
          gh_hbm, gt_hbm, eh_hbm, et_hbm, rr_hbm,
          idxh_v, idxt_v, idxr_v, rows_v, small_v, *sems):
        gsems = sems[:D]
        ssems = sems[D:2 * D]
        msem = sems[2 * D]
        wid = lax.axis_index("s") * NC + lax.axis_index("c")
        base = wid * ROWS_W

        pltpu.sync_copy(h_hbm.at[pl.ds(base, ROWS_W)], idxh_v)
        pltpu.sync_copy(t_hbm.at[pl.ds(base, ROWS_W)], idxt_v)
        pltpu.sync_copy(r_hbm.at[pl.ds(base, ROWS_W)], idxr_v)

        work = []
        for idx_v, out_hbm in ((idxh_v, gh_hbm), (idxt_v, gt_hbm)):
            for c in range(NCH):
                work.append((idx_v, out_hbm, c))
        n = len(work)

        def start_gather(i):
            idx_v, _, c = work[i]
            d = pltpu.make_async_copy(
                vis_hbm.at[idx_v.at[pl.ds(c * CH, CH)]],
                rows_v.at[i % D], gsems[i % D])
            d.start()
            return d

        def start_scatter(i):
            _, out_hbm, c = work[i]
            d = pltpu.make_async_copy(
                rows_v.at[i % D],
                out_hbm.at[pl.ds(base + c * CH, CH)], ssems[i % D])
            d.start()
            return d

        pg = [None] * n
        ps = [None] * n
        for i in range(D - 1):
            pg[i] = start_gather(i)

        for idx_v, src, dst in ((idxh_v, ent_hbm, eh_hbm),
                                (idxt_v, ent_hbm, et_hbm),
                                (idxr_v, rel_hbm, rr_hbm)):
            pltpu.async_copy(src.at[idx_v], small_v, msem).wait()
            pltpu.sync_copy(small_v, dst.at[pl.ds(base, ROWS_W)])

        for i in range(n):
            j = i + D - 1
            if j < n:
                if i >= 1:
                    ps[i - 1].wait()
                pg[j] = start_gather(j)
            pg[i].wait()
            ps[i] = start_scatter(i)
        for k in range(n - D, n):
            ps[k].wait()

    return k(batch_h, batch_t, batch_r, ent_emb, rel_emb, vis_emb)


def _tc_body(gh_ref, gt_ref, eh_ref, et_ref, rr_ref, mode_ref,
             wp_ref, bp_ref, wi_ref, bi_ref, out_ref):
    f32 = jnp.float32

    def proj(x, w, b):
        y = lax.dot_general(x, w[...], (((1,), (1,)), ((), ())),
                            preferred_element_type=f32)
        return y + b[...]

    def normalize(x):
        n = jnp.sqrt(jnp.sum(x * x, axis=-1, keepdims=True))
        return x / jnp.maximum(n, 1e-12)

    he = normalize(proj(eh_ref[...], wp_ref, bp_ref))
    te = normalize(proj(et_ref[...], wp_ref, bp_ref))
    hv = normalize(proj(gh_ref[...], wi_ref, bi_ref))
    tv = normalize(proj(gt_ref[...], wi_ref, bi_ref))
    rn = normalize(rr_ref[...])

    def l1(h, t):
        return jnp.sum(jnp.abs(h + rn - t), axis=-1)

    tt = l1(he, te)
    ii = l1(hv, tv)
    ti = l1(he, tv)
    it = l1(hv, te)

    mode = mode_ref[0, 0, :]
    score = (jnp.where(mode == 0, tt, 0.0)
             + jnp.where(mode == 1, it + ti, 0.0)
             + jnp.where(mode == 2, ii, 0.0))
    out_ref[0, 0, :] = score


def _tc_compute(gh, gt, eh, et, rr, task_mode, W_proj, b_proj, W_img, b_img):
    nb = gh.shape[0]
    NT = nb // TB
    mode3 = task_mode.astype(jnp.int32).reshape(NT, 1, TB)
    bp = b_proj.reshape(1, DIM)
    bi = b_img.reshape(1, DIM)
    grid = (NT,)
    out = pl.pallas_call(
        _tc_body,
        grid=grid,
        in_specs=[
            pl.BlockSpec((TB, VIS), lambda i: (i, 0)),
            pl.BlockSpec((TB, VIS), lambda i: (i, 0)),
            pl.BlockSpec((TB, DIM), lambda i: (i, 0)),
            pl.BlockSpec((TB, DIM), lambda i: (i, 0)),
            pl.BlockSpec((TB, DIM), lambda i: (i, 0)),
            pl.BlockSpec((1, 1, TB), lambda i: (i, 0, 0)),
            pl.BlockSpec((DIM, DIM), lambda i: (0, 0)),
            pl.BlockSpec((1, DIM), lambda i: (0, 0)),
            pl.BlockSpec((DIM, VIS), lambda i: (0, 0)),
            pl.BlockSpec((1, DIM), lambda i: (0, 0)),
        ],
        out_specs=pl.BlockSpec((1, 1, TB), lambda i: (i, 0, 0)),
        out_shape=jax.ShapeDtypeStruct((NT, 1, TB), jnp.float32),
    )(gh, gt, eh, et, rr, mode3, W_proj, bp, W_img, bi)
    return out.reshape(nb)


def kernel(batch_h, batch_t, batch_r, task_mode, ent_emb, rel_emb, vis_emb,
           W_proj, b_proj, W_img, b_img):
    h = batch_h.astype(jnp.int32)
    t = batch_t.astype(jnp.int32)
    r = batch_r.astype(jnp.int32)
    outs = []
    for g in range(G):
        sl = slice(g * CB, (g + 1) * CB)
        gh, gt, eh, et, rr = _sc_gather(h[sl], t[sl], r[sl],
                                        ent_emb, rel_emb, vis_emb)
        outs.append(_tc_compute(gh, gt, eh, et, rr, task_mode[sl],
                                W_proj, b_proj, W_img, b_img))
    return jnp.concatenate(outs)


# speedup vs baseline: 1.7992x; 1.7992x over previous
import functools

import jax
import jax.numpy as jnp
from jax import lax
from jax.experimental import pallas as pl
from jax.experimental.pallas import tpu as pltpu
from jax.experimental.pallas import tpu_sc as plsc

ENT = 100000
REL = 1000
DIM = 128
VIS = 4096
B = 4096

NC = 2
NS = 16
NW = NC * NS
ROWS_W = B // NW

TB = 256
NT = B // TB


def _sc_gather_small(batch_h, batch_t, batch_r, ent_emb, rel_emb):
    mesh = plsc.VectorSubcoreMesh(core_axis_name="c", subcore_axis_name="s")

    @functools.partial(
        pl.kernel,
        out_type=(
            jax.ShapeDtypeStruct((B, DIM), jnp.float32),
            jax.ShapeDtypeStruct((B, DIM), jnp.float32),
            jax.ShapeDtypeStruct((B, DIM), jnp.float32),
        ),
        mesh=mesh,
        scratch_types=[
            pltpu.VMEM((ROWS_W,), jnp.int32),
            pltpu.VMEM((ROWS_W,), jnp.int32),
            pltpu.VMEM((ROWS_W,), jnp.int32),
            pltpu.VMEM((ROWS_W, DIM), jnp.float32),
            pltpu.VMEM((ROWS_W, DIM), jnp.float32),
            pltpu.VMEM((ROWS_W, DIM), jnp.float32),
            pltpu.SemaphoreType.DMA,
            pltpu.SemaphoreType.DMA,
            pltpu.SemaphoreType.DMA,
        ],
    )
    def k(h_hbm, t_hbm, r_hbm, ent_hbm, rel_hbm,
          eh_hbm, et_hbm, rr_hbm,
          idxh_v, idxt_v, idxr_v, bh_v, bt_v, br_v, s0, s1, s2):
        wid = lax.axis_index("s") * NC + lax.axis_index("c")
        base = wid * ROWS_W

        pltpu.sync_copy(h_hbm.at[pl.ds(base, ROWS_W)], idxh_v)
        pltpu.sync_copy(t_hbm.at[pl.ds(base, ROWS_W)], idxt_v)
        pltpu.sync_copy(r_hbm.at[pl.ds(base, ROWS_W)], idxr_v)

        g0 = pltpu.make_async_copy(ent_hbm.at[idxh_v], bh_v, s0)
        g1 = pltpu.make_async_copy(ent_hbm.at[idxt_v], bt_v, s1)
        g2 = pltpu.make_async_copy(rel_hbm.at[idxr_v], br_v, s2)
        g0.start(); g1.start(); g2.start()
        g0.wait()
        w0 = pltpu.make_async_copy(bh_v, eh_hbm.at[pl.ds(base, ROWS_W)], s0)
        w0.start()
        g1.wait()
        w1 = pltpu.make_async_copy(bt_v, et_hbm.at[pl.ds(base, ROWS_W)], s1)
        w1.start()
        g2.wait()
        w2 = pltpu.make_async_copy(br_v, rr_hbm.at[pl.ds(base, ROWS_W)], s2)
        w2.start()
        w0.wait(); w1.wait(); w2.wait()

    return k(batch_h, batch_t, batch_r, ent_emb, rel_emb)


def _tc_body(idxh_s, idxt_s,
             vis_ref, eh_ref, et_ref, rr_ref, mode_ref,
             wp_ref, bp_ref, wi_ref, bi_ref, out_ref,
             vh_buf, vt_buf, hsem, tsem):
    i = pl.program_id(0)
    f32 = jnp.float32

    def issue(block, slot):
        base = block * TB

        def body(j, _):
            pltpu.make_async_copy(
                vis_ref.at[idxh_s[base + j]],
                vh_buf.at[slot * TB + j], hsem.at[slot]).start()
            pltpu.make_async_copy(
                vis_ref.at[idxt_s[base + j]],
                vt_buf.at[slot * TB + j], tsem.at[slot]).start()
            return 0

        lax.fori_loop(0, TB, body, 0, unroll=8)

    def wait_slot(slot):
        pltpu.make_async_copy(
            vis_ref.at[pl.ds(0, TB)],
            vh_buf.at[pl.ds(slot * TB, TB)], hsem.at[slot]).wait()
        pltpu.make_async_copy(
            vis_ref.at[pl.ds(0, TB)],
            vt_buf.at[pl.ds(slot * TB, TB)], tsem.at[slot]).wait()

    @pl.when(i == 0)
    def _():
        issue(0, 0)

    @pl.when(i + 1 < NT)
    def _():
        issue(i + 1, (i + 1) % 2)

    slot = i % 2
    wait_slot(slot)
    gh = vh_buf[pl.ds(slot * TB, TB), :]
    gt = vt_buf[pl.ds(slot * TB, TB), :]

    def proj(x, w, b):
        y = lax.dot_general(x, w[...], (((1,), (1,)), ((), ())),
                            preferred_element_type=f32)
        return y + b[...]

    def normalize(x):
        n = jnp.sqrt(jnp.sum(x * x, axis=-1, keepdims=True))
        return x / jnp.maximum(n, 1e-12)

    he = normalize(proj(eh_ref[...], wp_ref, bp_ref))
    te = normalize(proj(et_ref[...], wp_ref, bp_ref))
    hv = normalize(proj(gh, wi_ref, bi_ref))
    tv = normalize(proj(gt, wi_ref, bi_ref))
    rn = normalize(rr_ref[...])

    def l1(h, t):
        return jnp.sum(jnp.abs(h + rn - t), axis=-1)

    tt = l1(he, te)
    ii = l1(hv, tv)
    ti = l1(he, tv)
    it = l1(hv, te)

    mode = mode_ref[0, 0, :]
    score = (jnp.where(mode == 0, tt, 0.0)
             + jnp.where(mode == 1, it + ti, 0.0)
             + jnp.where(mode == 2, ii, 0.0))
    out_ref[0, 0, :] = score


def _tc_compute(idx_h, idx_t, vis_emb, eh, et, rr, task_mode,
                W_proj, b_proj, W_img, b_img):
    mode3 = task_mode.astype(jnp.int32).reshape(NT, 1, TB)
    bp = b_proj.reshape(1, DIM)
    bi = b_img.reshape(1, DIM)

    def blk(i, *_):
        return (i, 0)

    def blk3(i, *_):
        return (i, 0, 0)

    def const2(i, *_):
        return (0, 0)

    grid_spec = pltpu.PrefetchScalarGridSpec(
        num_scalar_prefetch=2,
        grid=(NT,),
        in_specs=[
            pl.BlockSpec(memory_space=pl.ANY),
            pl.BlockSpec((TB, DIM), blk),
            pl.BlockSpec((TB, DIM), blk),
            pl.BlockSpec((TB, DIM), blk),
            pl.BlockSpec((1, 1, TB), blk3),
            pl.BlockSpec((DIM, DIM), const2),
            pl.BlockSpec((1, DIM), const2),
            pl.BlockSpec((DIM, VIS), const2),
            pl.BlockSpec((1, DIM), const2),
        ],
        out_specs=pl.BlockSpec((1, 1, TB), blk3),
        scratch_shapes=[
            pltpu.VMEM((2 * TB, VIS), jnp.float32),
            pltpu.VMEM((2 * TB, VIS), jnp.float32),
            pltpu.SemaphoreType.DMA((2,)),
            pltpu.SemaphoreType.DMA((2,)),
        ],
    )
    out = pl.pallas_call(
        _tc_body,
        grid_spec=grid_spec,
        out_shape=jax.ShapeDtypeStruct((NT, 1, TB), jnp.float32),
    )(idx_h, idx_t, vis_emb, eh, et, rr, mode3, W_proj, bp, W_img, bi)
    return out.reshape(B)


def kernel(batch_h, batch_t, batch_r, task_mode, ent_emb, rel_emb, vis_emb,
           W_proj, b_proj, W_img, b_img):
    h = batch_h.astype(jnp.int32)
    t = batch_t.astype(jnp.int32)
    r = batch_r.astype(jnp.int32)
    eh, et, rr = _sc_gather_small(h, t, r, ent_emb, rel_emb)
    return _tc_compute(h, t, vis_emb, eh, et, rr, task_mode,
                       W_proj, b_proj, W_img, b_img)


# speedup vs baseline: 1.8284x; 1.0162x over previous
import functools

import jax
import jax.numpy as jnp
from jax import lax
from jax.experimental import pallas as pl
from jax.experimental.pallas import tpu as pltpu
from jax.experimental.pallas import tpu_sc as plsc

ENT = 100000
REL = 1000
DIM = 128
VIS = 4096
B = 4096

NC = 2
NS = 16
NW = NC * NS
ROWS_W = B // NW

TB = 512
NT = B // TB


def _sc_gather_small(batch_h, batch_t, batch_r, ent_emb, rel_emb):
    mesh = plsc.VectorSubcoreMesh(core_axis_name="c", subcore_axis_name="s")

    @functools.partial(
        pl.kernel,
        out_type=(
            jax.ShapeDtypeStruct((B, DIM), jnp.float32),
            jax.ShapeDtypeStruct((B, DIM), jnp.float32),
            jax.ShapeDtypeStruct((B, DIM), jnp.float32),
        ),
        mesh=mesh,
        scratch_types=[
            pltpu.VMEM((ROWS_W,), jnp.int32),
            pltpu.VMEM((ROWS_W,), jnp.int32),
            pltpu.VMEM((ROWS_W,), jnp.int32),
            pltpu.VMEM((ROWS_W, DIM), jnp.float32),
            pltpu.VMEM((ROWS_W, DIM), jnp.float32),
            pltpu.VMEM((ROWS_W, DIM), jnp.float32),
            pltpu.SemaphoreType.DMA,
            pltpu.SemaphoreType.DMA,
            pltpu.SemaphoreType.DMA,
        ],
    )
    def k(h_hbm, t_hbm, r_hbm, ent_hbm, rel_hbm,
          eh_hbm, et_hbm, rr_hbm,
          idxh_v, idxt_v, idxr_v, bh_v, bt_v, br_v, s0, s1, s2):
        wid = lax.axis_index("s") * NC + lax.axis_index("c")
        base = wid * ROWS_W

        pltpu.sync_copy(h_hbm.at[pl.ds(base, ROWS_W)], idxh_v)
        pltpu.sync_copy(t_hbm.at[pl.ds(base, ROWS_W)], idxt_v)
        pltpu.sync_copy(r_hbm.at[pl.ds(base, ROWS_W)], idxr_v)

        g0 = pltpu.make_async_copy(ent_hbm.at[idxh_v], bh_v, s0)
        g1 = pltpu.make_async_copy(ent_hbm.at[idxt_v], bt_v, s1)
        g2 = pltpu.make_async_copy(rel_hbm.at[idxr_v], br_v, s2)
        g0.start(); g1.start(); g2.start()
        g0.wait()
        w0 = pltpu.make_async_copy(bh_v, eh_hbm.at[pl.ds(base, ROWS_W)], s0)
        w0.start()
        g1.wait()
        w1 = pltpu.make_async_copy(bt_v, et_hbm.at[pl.ds(base, ROWS_W)], s1)
        w1.start()
        g2.wait()
        w2 = pltpu.make_async_copy(br_v, rr_hbm.at[pl.ds(base, ROWS_W)], s2)
        w2.start()
        w0.wait(); w1.wait(); w2.wait()

    return k(batch_h, batch_t, batch_r, ent_emb, rel_emb)


def _tc_body(idxh_s, idxt_s,
             vis_ref, eh_ref, et_ref, rr_ref, mode_ref,
             wp_ref, bp_ref, wi_ref, bi_ref, out_ref,
             vh_buf, vt_buf, hsem, tsem):
    i = pl.program_id(0)
    f32 = jnp.float32

    def issue(block, slot):
        base = block * TB

        def body(j, _):
            pltpu.make_async_copy(
                vis_ref.at[idxh_s[base + j]],
                vh_buf.at[slot * TB + j], hsem.at[slot]).start()
            pltpu.make_async_copy(
                vis_ref.at[idxt_s[base + j]],
                vt_buf.at[slot * TB + j], tsem.at[slot]).start()
            return 0

        lax.fori_loop(0, TB, body, 0, unroll=32)

    def wait_slot(slot):
        pltpu.make_async_copy(
            vis_ref.at[pl.ds(0, TB)],
            vh_buf.at[pl.ds(slot * TB, TB)], hsem.at[slot]).wait()
        pltpu.make_async_copy(
            vis_ref.at[pl.ds(0, TB)],
            vt_buf.at[pl.ds(slot * TB, TB)], tsem.at[slot]).wait()

    @pl.when(i == 0)
    def _():
        issue(0, 0)

    @pl.when(i + 1 < NT)
    def _():
        issue(i + 1, (i + 1) % 2)

    slot = i % 2
    wait_slot(slot)
    gh = vh_buf[pl.ds(slot * TB, TB), :]
    gt = vt_buf[pl.ds(slot * TB, TB), :]

    def proj(x, w, b):
        y = lax.dot_general(x, w[...], (((1,), (1,)), ((), ())),
                            preferred_element_type=f32)
        return y + b[...]

    def normalize(x):
        n = jnp.sqrt(jnp.sum(x * x, axis=-1, keepdims=True))
        return x / jnp.maximum(n, 1e-12)

    he = normalize(proj(eh_ref[...], wp_ref, bp_ref))
    te = normalize(proj(et_ref[...], wp_ref, bp_ref))
    hv = normalize(proj(gh, wi_ref, bi_ref))
    tv = normalize(proj(gt, wi_ref, bi_ref))
    rn = normalize(rr_ref[...])

    def l1(h, t):
        return jnp.sum(jnp.abs(h + rn - t), axis=-1)

    tt = l1(he, te)
    ii = l1(hv, tv)
    ti = l1(he, tv)
    it = l1(hv, te)

    mode = mode_ref[0, 0, :]
    score = (jnp.where(mode == 0, tt, 0.0)
             + jnp.where(mode == 1, it + ti, 0.0)
             + jnp.where(mode == 2, ii, 0.0))
    out_ref[0, 0, :] = score


def _tc_compute(idx_h, idx_t, vis_emb, eh, et, rr, task_mode,
                W_proj, b_proj, W_img, b_img):
    mode3 = task_mode.astype(jnp.int32).reshape(NT, 1, TB)
    bp = b_proj.reshape(1, DIM)
    bi = b_img.reshape(1, DIM)

    def blk(i, *_):
        return (i, 0)

    def blk3(i, *_):
        return (i, 0, 0)

    def const2(i, *_):
        return (0, 0)

    grid_spec = pltpu.PrefetchScalarGridSpec(
        num_scalar_prefetch=2,
        grid=(NT,),
        in_specs=[
            pl.BlockSpec(memory_space=pl.ANY),
            pl.BlockSpec((TB, DIM), blk),
            pl.BlockSpec((TB, DIM), blk),
            pl.BlockSpec((TB, DIM), blk),
            pl.BlockSpec((1, 1, TB), blk3),
            pl.BlockSpec((DIM, DIM), const2),
            pl.BlockSpec((1, DIM), const2),
            pl.BlockSpec((DIM, VIS), const2),
            pl.BlockSpec((1, DIM), const2),
        ],
        out_specs=pl.BlockSpec((1, 1, TB), blk3),
        scratch_shapes=[
            pltpu.VMEM((2 * TB, VIS), jnp.float32),
            pltpu.VMEM((2 * TB, VIS), jnp.float32),
            pltpu.SemaphoreType.DMA((2,)),
            pltpu.SemaphoreType.DMA((2,)),
        ],
    )
    out = pl.pallas_call(
        _tc_body,
        grid_spec=grid_spec,
        out_shape=jax.ShapeDtypeStruct((NT, 1, TB), jnp.float32),
    )(idx_h, idx_t, vis_emb, eh, et, rr, mode3, W_proj, bp, W_img, bi)
    return out.reshape(B)


def kernel(batch_h, batch_t, batch_r, task_mode, ent_emb, rel_emb, vis_emb,
           W_proj, b_proj, W_img, b_img):
    h = batch_h.astype(jnp.int32)
    t = batch_t.astype(jnp.int32)
    r = batch_r.astype(jnp.int32)
    eh, et, rr = _sc_gather_small(h, t, r, ent_emb, rel_emb)
    return _tc_compute(h, t, vis_emb, eh, et, rr, task_mode,
                       W_proj, b_proj, W_img, b_img)
